# baseline (speedup 1.0000x reference)
import functools

import jax
import jax.numpy as jnp
from jax.experimental import pallas as pl
from jax.experimental.pallas import tpu as pltpu
from jax.experimental.pallas import tpu_sc as plsc

_CH = 128


@jax.jit
def _biased_embedding(index, vect_weight, bias_weight):
    (B,) = index.shape
    V, D = vect_weight.shape
    info = plsc.get_sparse_core_info()
    nw = info.num_cores * info.num_subcores
    b_per_w = B // nw
    nch = b_per_w // _CH
    mesh = plsc.VectorSubcoreMesh(core_axis_name="core", subcore_axis_name="subcore")
    idx2d = index.reshape(B // _CH, _CH)

    scratch = (
        [pltpu.VMEM((nch, _CH), jnp.int32)]
        + [pltpu.VMEM((_CH, D), jnp.float32) for _ in range(nch)]
        + [pltpu.VMEM((_CH,), jnp.float32) for _ in range(nch)]
        + [pltpu.SemaphoreType.DMA]
    )

    @functools.partial(
        pl.kernel,
        out_type=(
            jax.ShapeDtypeStruct((B,), jnp.float32),
            jax.ShapeDtypeStruct((B, D), jnp.float32),
        ),
        mesh=mesh,
        scratch_types=scratch,
        compiler_params=pltpu.CompilerParams(use_tc_tiling_on_sc=False),
    )
    def run(vect_hbm, bias_hbm, idx_hbm, bias_out, vect_out, idx_v, *bufs):
        rows = bufs[:nch]
        bvals = bufs[nch:2 * nch]
        sem = bufs[2 * nch]
        wid = jax.lax.axis_index("subcore") * info.num_cores + jax.lax.axis_index("core")
        base = wid * b_per_w

        pltpu.sync_copy(idx_hbm.at[pl.ds(wid * nch, nch)], idx_v)
        copies = []
        for j in range(nch):
            copies.append(pltpu.async_copy(vect_hbm.at[idx_v.at[j]], rows[j], sem))
            copies.append(pltpu.async_copy(bias_hbm.at[idx_v.at[j]], bvals[j], sem))
        for c in copies:
            c.wait()
        for j in range(nch):
            pltpu.sync_copy(rows[j], vect_out.at[pl.ds(base + j * _CH, _CH)])
            pltpu.sync_copy(bvals[j], bias_out.at[pl.ds(base + j * _CH, _CH)])

    bias, vect = run(vect_weight, bias_weight.reshape(V), idx2d)
    return bias, vect


def kernel(index, vect_weight, bias_weight):
    return _biased_embedding(index.astype(jnp.int32), vect_weight, bias_weight)
